# baseline (speedup 1.0000x reference)
import functools

import jax
import jax.numpy as jnp
from jax import lax
from jax.experimental import pallas as pl
from jax.experimental.pallas import tpu as pltpu
from jax.experimental.pallas import tpu_sc as plsc

N = 10000
D = 128
E = 320000
E2 = E + N
NC = 2
NS = 16
NW = NC * NS
CH = 64
EPT = 10752
NCH = EPT // CH
NO = N + 8


_segsum_kernel_kwargs = dict(
    out_type=jax.ShapeDtypeStruct((NO, D), jnp.float32),
    scratch_types=[
        pltpu.VMEM((NCH, 1, CH), jnp.int32),
        pltpu.VMEM((NCH, 1, CH), jnp.int32),
        pltpu.VMEM((EPT,), jnp.float32),
        pltpu.VMEM((EPT,), jnp.float32),
        pltpu.VMEM((2, CH, D), jnp.float32),
        pltpu.SemaphoreType.DMA,
        pltpu.SemaphoreType.DMA,
        pltpu.SemaphoreType.DMA,
        pltpu.SemaphoreType.DMA,
    ],
    compiler_params=pltpu.CompilerParams(needs_layout_passes=False,
                                         use_tc_tiling_on_sc=False),
)


def _segsum_body(h_hbm, src_hbm, sidx_hbm, w_hbm, keep_hbm, out_hbm,
                 src_v, sidx_v, w_v, keep_v, rows_v,
                 sem0, sem1, sem2, sem3):
    c = lax.axis_index("c")
    s = lax.axis_index("s")
    wid = s * NC + c
    sems_g = [sem0, sem1]
    sems_s = [sem2, sem3]

    pltpu.sync_copy(src_hbm.at[pl.ds(wid * NCH, NCH)], src_v)
    pltpu.sync_copy(sidx_hbm.at[pl.ds(wid * NCH, NCH)], sidx_v)
    pltpu.sync_copy(w_hbm.at[wid, 0], w_v)
    pltpu.sync_copy(keep_hbm.at[wid, 0], keep_v)

    pltpu.async_copy(h_hbm.at[src_v.at[0, 0]], rows_v.at[0], sems_g[0])

    acc0 = tuple(jnp.zeros((16,), jnp.float32) for _ in range(D // 16))

    @pl.loop(0, NCH, step=2, init_carry=acc0)
    def _chunks(i, acc):
        for b in range(2):
            j = i + b
            nxt = j + 1

            @pl.when(nxt < NCH)
            def _():
                @pl.when(j >= 1)
                def _():
                    pltpu.make_async_copy(
                        rows_v.at[1 - b],
                        out_hbm.at[sidx_v.at[j, 0]],
                        sems_s[1 - b]).wait()
                pltpu.async_copy(h_hbm.at[src_v.at[nxt, 0]],
                                 rows_v.at[1 - b], sems_g[1 - b])

            pltpu.make_async_copy(h_hbm.at[src_v.at[j, 0]],
                                  rows_v.at[b], sems_g[b]).wait()

            def _scale(e):
                wv = plsc.load_gather(w_v, [jnp.full((16,), 0, jnp.int32)
                                            + j * CH + e])
                for g in range(D // 16):
                    sl = pl.ds(g * 16, 16)
                    rows_v[b, e, sl] = rows_v[b, e, sl] * wv
            plsc.parallel_loop(0, CH, 1, unroll=4)(_scale)

            def _accum(e, a):
                kv = plsc.load_gather(keep_v, [jnp.full((16,), 0, jnp.int32)
                                               + j * CH + e])
                m = kv > 0.5
                new = []
                for g in range(D // 16):
                    sl = pl.ds(g * 16, 16)
                    v = jnp.where(m, a[g], 0.0) + rows_v[b, e, sl]
                    rows_v[b, e, sl] = v
                    new.append(v)
                return tuple(new)
            acc = lax.fori_loop(0, CH, _accum, acc)

            pltpu.async_copy(rows_v.at[b], out_hbm.at[sidx_v.at[j, 0]],
                             sems_s[b])
        return acc

    for b in range(2):
        pltpu.make_async_copy(rows_v.at[b], out_hbm.at[sidx_v.at[0, 0]],
                              sems_s[b]).wait()


_segsum_cache = []


def _segsum(*args):
    if not _segsum_cache:
        mesh = plsc.VectorSubcoreMesh(core_axis_name="c", subcore_axis_name="s",
                                      num_cores=NC, num_subcores=NS)
        _segsum_cache.append(functools.partial(
            pl.kernel, mesh=mesh, **_segsum_kernel_kwargs)(_segsum_body))
    return _segsum_cache[0](*args)


_RB = 2000


def _layer_mid_body(agg_ref, w_ref, b_ref, o_ref):
    o_ref[...] = jnp.tanh(
        jnp.dot(agg_ref[...], w_ref[...], preferred_element_type=jnp.float32,
                precision=jax.lax.Precision.DEFAULT)
        + b_ref[...])


def _layer_last_body(agg_ref, w_ref, b_ref, y_ref, dt_ref, o_ref):
    f = jnp.dot(agg_ref[...], w_ref[...], preferred_element_type=jnp.float32,
                precision=jax.lax.Precision.DEFAULT) + b_ref[...]
    o_ref[...] = y_ref[...] + dt_ref[0, 0] * f


_grid = (N // _RB,)
_agg_spec = pl.BlockSpec((_RB, D), lambda i: (i, 0))
_w_spec = pl.BlockSpec((D, D), lambda i: (0, 0))
_b_spec = pl.BlockSpec((1, D), lambda i: (0, 0))
_row_spec = pl.BlockSpec((_RB, D), lambda i: (i, 0))
_dt_spec = pl.BlockSpec((1, 1), lambda i: (0, 0))
_out_sds = jax.ShapeDtypeStruct((N, D), jnp.float32)

_layer_mid = pl.pallas_call(
    _layer_mid_body, grid=_grid, out_shape=_out_sds,
    in_specs=[_agg_spec, _w_spec, _b_spec], out_specs=_row_spec)

_layer_last = pl.pallas_call(
    _layer_last_body, grid=_grid, out_shape=_out_sds,
    in_specs=[_agg_spec, _w_spec, _b_spec, _row_spec, _dt_spec],
    out_specs=_row_spec)


def _build_edge_plan(edge_index, edge_weight):
    src2 = jnp.concatenate([edge_index[0],
                            jnp.zeros((N,), jnp.int32)])
    dst2 = jnp.concatenate([edge_index[1],
                            jnp.arange(N, dtype=jnp.int32)])
    w2 = jnp.concatenate([edge_weight, jnp.zeros((N,), jnp.float32)])
    order = jnp.argsort(dst2, stable=True)
    ss, dd, ww = src2[order], dst2[order], w2[order]

    iot = jnp.arange(E2, dtype=jnp.int32)
    neq = dd[1:] != dd[:-1]
    is_start = jnp.concatenate([jnp.array([True]), neq])
    is_end = jnp.concatenate([neq, jnp.array([True])])
    seg_start_pos = jax.lax.cummax(jnp.where(is_start, iot, 0))

    targets = (jnp.arange(NW, dtype=jnp.int32) * (E2 // NW))
    starts = seg_start_pos[targets]
    ends = jnp.concatenate([starts[1:], jnp.array([E2], jnp.int32)])

    k = jnp.arange(EPT, dtype=jnp.int32)
    ge = starts[:, None] + k[None, :]
    valid = ge < ends[:, None]
    gec = jnp.minimum(ge, E2 - 1)
    src_t = jnp.where(valid, ss[gec], 0)
    w_t = jnp.where(valid, ww[gec], 0.0)
    keep_t = jnp.where(valid & is_start[gec], 0.0, 1.0)
    sidx_t = jnp.where(valid & is_end[gec], dd[gec], N)

    return (src_t.reshape(NW * NCH, 1, CH).astype(jnp.int32),
            sidx_t.reshape(NW * NCH, 1, CH).astype(jnp.int32),
            w_t.reshape(NW, 1, EPT),
            keep_t.reshape(NW, 1, EPT))


def kernel(z_t0_nodes, t_eval_points, edge_index, edge_weight,
           W0, b0, W1, b1, W2, b2):
    srcr, sidxr, wr, keepr = _build_edge_plan(edge_index, edge_weight)

    Ws = [W0, W1, W2]
    bs = [b0.reshape(1, D), b1.reshape(1, D), b2.reshape(1, D)]

    ys = [z_t0_nodes]
    y = z_t0_nodes
    T = t_eval_points.shape[0]
    n_sub = 4
    for i in range(T - 1):
        dt = ((t_eval_points[i + 1] - t_eval_points[i]) / n_sub).reshape(1, 1)
        for _ in range(n_sub):
            h = y
            for l in range(2):
                agg = _segsum(h, srcr, sidxr, wr, keepr)
                h = _layer_mid(agg, Ws[l], bs[l])
            agg = _segsum(h, srcr, sidxr, wr, keepr)
            y = _layer_last(agg, Ws[2], bs[2], y, dt)
        ys.append(y)
    return jnp.stack(ys, axis=0)


# speedup vs baseline: 9.2226x; 9.2226x over previous
import functools

import jax
import jax.numpy as jnp
from jax import lax
from jax.experimental import pallas as pl
from jax.experimental.pallas import tpu as pltpu
from jax.experimental.pallas import tpu_sc as plsc

N = 10000
D = 128
E = 320000
E2 = E + N
NC = 2
NS = 16
NW = NC * NS
CH = 64
EPT = 10752
NCH = EPT // CH
NO = N + NW


_segsum_kernel_kwargs = dict(
    out_type=jax.ShapeDtypeStruct((NO, D), jnp.float32),
    scratch_types=[
        pltpu.VMEM((NCH, 1, CH), jnp.int32),
        pltpu.VMEM((NCH, 1, CH), jnp.int32),
        pltpu.VMEM((EPT,), jnp.float32),
        pltpu.VMEM((EPT,), jnp.float32),
        pltpu.VMEM((2, CH, D), jnp.float32),
        pltpu.SemaphoreType.DMA,
        pltpu.SemaphoreType.DMA,
        pltpu.SemaphoreType.DMA,
        pltpu.SemaphoreType.DMA,
    ],
    compiler_params=pltpu.CompilerParams(needs_layout_passes=False,
                                         use_tc_tiling_on_sc=False),
)


def _segsum_body(h_hbm, src_hbm, sidx_hbm, w_hbm, keep_hbm, out_hbm,
                 src_v, sidx_v, w_v, keep_v, rows_v,
                 sem0, sem1, sem2, sem3):
    c = lax.axis_index("c")
    s = lax.axis_index("s")
    wid = s * NC + c
    sems_g = [sem0, sem1]
    sems_s = [sem2, sem3]

    pltpu.sync_copy(src_hbm.at[pl.ds(wid * NCH, NCH)], src_v)
    pltpu.sync_copy(sidx_hbm.at[pl.ds(wid * NCH, NCH)], sidx_v)
    pltpu.sync_copy(w_hbm.at[wid, 0], w_v)
    pltpu.sync_copy(keep_hbm.at[wid, 0], keep_v)

    pltpu.async_copy(h_hbm.at[src_v.at[0, 0]], rows_v.at[0], sems_g[0])

    acc0 = tuple(jnp.zeros((16,), jnp.float32) for _ in range(D // 16))

    @pl.loop(0, NCH, step=2, init_carry=acc0)
    def _chunks(i, acc):
        for b in range(2):
            j = i + b
            nxt = j + 1

            @pl.when(nxt < NCH)
            def _():
                @pl.when(j >= 1)
                def _():
                    pltpu.make_async_copy(
                        rows_v.at[1 - b],
                        out_hbm.at[sidx_v.at[j, 0]],
                        sems_s[1 - b]).wait()
                pltpu.async_copy(h_hbm.at[src_v.at[nxt, 0]],
                                 rows_v.at[1 - b], sems_g[1 - b])

            pltpu.make_async_copy(h_hbm.at[src_v.at[j, 0]],
                                  rows_v.at[b], sems_g[b]).wait()

            def _scale(e):
                wv = plsc.load_gather(w_v, [jnp.full((16,), 0, jnp.int32)
                                            + j * CH + e])
                for g in range(D // 16):
                    sl = pl.ds(g * 16, 16)
                    rows_v[b, e, sl] = rows_v[b, e, sl] * wv
            plsc.parallel_loop(0, CH, 1, unroll=4)(_scale)

            def _accum(e, a):
                kv = plsc.load_gather(keep_v, [jnp.full((16,), 0, jnp.int32)
                                               + j * CH + e])
                m = kv > 0.5
                new = []
                for g in range(D // 16):
                    sl = pl.ds(g * 16, 16)
                    v = jnp.where(m, a[g], 0.0) + rows_v[b, e, sl]
                    rows_v[b, e, sl] = v
                    new.append(v)
                return tuple(new)
            acc = lax.fori_loop(0, CH, _accum, acc)

            pltpu.async_copy(rows_v.at[b], out_hbm.at[sidx_v.at[j, 0]],
                             sems_s[b])
        return acc

    for b in range(2):
        pltpu.make_async_copy(rows_v.at[b], out_hbm.at[sidx_v.at[0, 0]],
                              sems_s[b]).wait()


_segsum_cache = []


def _segsum(*args):
    if not _segsum_cache:
        mesh = plsc.VectorSubcoreMesh(core_axis_name="c", subcore_axis_name="s",
                                      num_cores=NC, num_subcores=NS)
        _segsum_cache.append(functools.partial(
            pl.kernel, mesh=mesh, **_segsum_kernel_kwargs)(_segsum_body))
    return _segsum_cache[0](*args)


_RB = 2000


def _layer_mid_body(agg_ref, w_ref, b_ref, o_ref):
    o_ref[...] = jnp.tanh(
        jnp.dot(agg_ref[...], w_ref[...], preferred_element_type=jnp.float32,
                precision=jax.lax.Precision.DEFAULT)
        + b_ref[...])


def _layer_last_body(agg_ref, w_ref, b_ref, y_ref, dt_ref, o_ref):
    f = jnp.dot(agg_ref[...], w_ref[...], preferred_element_type=jnp.float32,
                precision=jax.lax.Precision.DEFAULT) + b_ref[...]
    o_ref[...] = y_ref[...] + dt_ref[0, 0] * f


_grid = (N // _RB,)
_agg_spec = pl.BlockSpec((_RB, D), lambda i: (i, 0))
_w_spec = pl.BlockSpec((D, D), lambda i: (0, 0))
_b_spec = pl.BlockSpec((1, D), lambda i: (0, 0))
_row_spec = pl.BlockSpec((_RB, D), lambda i: (i, 0))
_dt_spec = pl.BlockSpec((1, 1), lambda i: (0, 0))
_out_sds = jax.ShapeDtypeStruct((N, D), jnp.float32)

_layer_mid = pl.pallas_call(
    _layer_mid_body, grid=_grid, out_shape=_out_sds,
    in_specs=[_agg_spec, _w_spec, _b_spec], out_specs=_row_spec)

_layer_last = pl.pallas_call(
    _layer_last_body, grid=_grid, out_shape=_out_sds,
    in_specs=[_agg_spec, _w_spec, _b_spec, _row_spec, _dt_spec],
    out_specs=_row_spec)


def _build_edge_plan(edge_index, edge_weight):
    src2 = jnp.concatenate([edge_index[0],
                            jnp.zeros((N,), jnp.int32)])
    dst2 = jnp.concatenate([edge_index[1],
                            jnp.arange(N, dtype=jnp.int32)])
    w2 = jnp.concatenate([edge_weight, jnp.zeros((N,), jnp.float32)])
    order = jnp.argsort(dst2, stable=True)
    ss, dd, ww = src2[order], dst2[order], w2[order]

    iot = jnp.arange(E2, dtype=jnp.int32)
    neq = dd[1:] != dd[:-1]
    is_start = jnp.concatenate([jnp.array([True]), neq])
    is_end = jnp.concatenate([neq, jnp.array([True])])
    seg_start_pos = jax.lax.cummax(jnp.where(is_start, iot, 0))

    targets = (jnp.arange(NW, dtype=jnp.int32) * (E2 // NW))
    starts = seg_start_pos[targets]
    ends = jnp.concatenate([starts[1:], jnp.array([E2], jnp.int32)])

    k = jnp.arange(EPT, dtype=jnp.int32)
    ge = starts[:, None] + k[None, :]
    valid = ge < ends[:, None]
    gec = jnp.minimum(ge, E2 - 1)
    src_t = jnp.where(valid, ss[gec], 0)
    w_t = jnp.where(valid, ww[gec], 0.0)
    keep_t = jnp.where(valid & is_start[gec], 0.0, 1.0)
    trash = N + jnp.arange(NW, dtype=jnp.int32)[:, None]
    sidx_t = jnp.where(valid & is_end[gec], dd[gec], trash)

    return (src_t.reshape(NW * NCH, 1, CH).astype(jnp.int32),
            sidx_t.reshape(NW * NCH, 1, CH).astype(jnp.int32),
            w_t.reshape(NW, 1, EPT),
            keep_t.reshape(NW, 1, EPT))


def kernel(z_t0_nodes, t_eval_points, edge_index, edge_weight,
           W0, b0, W1, b1, W2, b2):
    srcr, sidxr, wr, keepr = _build_edge_plan(edge_index, edge_weight)

    Ws = [W0, W1, W2]
    bs = [b0.reshape(1, D), b1.reshape(1, D), b2.reshape(1, D)]

    ys = [z_t0_nodes]
    y = z_t0_nodes
    T = t_eval_points.shape[0]
    n_sub = 4
    for i in range(T - 1):
        dt = ((t_eval_points[i + 1] - t_eval_points[i]) / n_sub).reshape(1, 1)
        for _ in range(n_sub):
            h = y
            for l in range(2):
                agg = _segsum(h, srcr, sidxr, wr, keepr)
                h = _layer_mid(agg, Ws[l], bs[l])
            agg = _segsum(h, srcr, sidxr, wr, keepr)
            y = _layer_last(agg, Ws[2], bs[2], y, dt)
        ys.append(y)
    return jnp.stack(ys, axis=0)
